# baseline (speedup 1.0000x reference)
import functools

import jax
import jax.numpy as jnp
import numpy as np
from jax.experimental import pallas as pl
from jax.experimental.pallas import tpu as pltpu

B = 16
C = 255
H = 64
W = 64
A = 3
CH = 85
HW = H * W
STRIDE = 8
_ANCHORS = np.array(
    [10, 13, 16, 30, 33, 23], dtype=np.float32
).reshape(3, 2) / float(STRIDE)
_AW = tuple(float(v) for v in (_ANCHORS[:, 0] / W))
_AH = tuple(float(v) for v in (_ANCHORS[:, 1] / H))

T = 4096


def _decode_kernel(x_ref, o_ref):
    i = pl.program_id(0)
    j = pl.program_id(1)
    a = i % A

    x2 = x_ref[0]
    sig = jax.nn.sigmoid(x2)

    rows = jax.lax.broadcasted_iota(jnp.int32, (8, T), 0)
    pos = jax.lax.broadcasted_iota(jnp.int32, (8, T), 1) + j * T
    gx = (pos % W).astype(jnp.float32)
    gy = (pos // W).astype(jnp.float32)
    g = jnp.where(rows == 0, gx, gy)

    aw = jnp.where(a == 0, _AW[0], jnp.where(a == 1, _AW[1], _AW[2]))
    ah = jnp.where(a == 0, _AH[0], jnp.where(a == 1, _AH[1], _AH[2]))
    sc = jnp.where(rows == 2, aw, ah)

    sig8 = sig[0:8]
    xy = (sig8 + g) * (1.0 / W)
    wh = jnp.exp(x2[0:8]) * sc
    top = jnp.where(rows < 2, xy, jnp.where(rows < 4, wh, sig8))
    assembled = jnp.concatenate([top, sig[8:CH]], axis=0)

    dt = assembled.T
    dt8 = dt.reshape(T // 8, 8, CH)
    v = jnp.concatenate([dt8[:, s, :] for s in range(8)], axis=1)
    o_ref[0] = v


@functools.partial(jax.jit, static_argnames=("interpret",))
def kernel(x, interpret: bool = False):
    xr = x.reshape(B * A, CH, HW)
    out = pl.pallas_call(
        _decode_kernel,
        grid=(B * A, HW // T),
        in_specs=[pl.BlockSpec((1, CH, T), lambda i, j: (i, 0, j))],
        out_specs=pl.BlockSpec((1, T // 8, 8 * CH), lambda i, j: (i, j, 0)),
        out_shape=jax.ShapeDtypeStruct((B * A, HW // 8, 8 * CH), jnp.float32),
        interpret=interpret,
    )(xr)
    return out.reshape(B, A * HW, CH)
